# baseline (speedup 1.0000x reference)
import functools

import jax
import jax.numpy as jnp
from jax import lax
from jax.experimental import pallas as pl
from jax.experimental.pallas import tpu as pltpu
from jax.experimental.pallas import tpu_sc as plsc

B = 16384
F = 32
N_ROWS = 1000000
PACK = 128 // F
NC = 2
NS = 16
L = 16
NW = NC * NS
BPW = B // NW
STAGES = 2
SPW = BPW // STAGES
SCHUNKS = SPW // L

_mesh = plsc.VectorSubcoreMesh(core_axis_name="c", subcore_axis_name="s")


@functools.partial(
    pl.kernel,
    mesh=_mesh,
    out_type=jax.ShapeDtypeStruct((B,), jnp.float32),
    compiler_params=pltpu.CompilerParams(
        needs_layout_passes=False, use_tc_tiling_on_sc=False),
    scratch_types=[
        pltpu.VMEM((BPW,), jnp.int32),
        pltpu.VMEM((BPW,), jnp.int32),
        pltpu.VMEM((BPW,), jnp.int32),
        pltpu.VMEM((BPW,), jnp.int32),
        pltpu.VMEM((BPW,), jnp.int32),
        pltpu.VMEM((BPW,), jnp.int32),
        pltpu.VMEM((SPW, PACK * F), jnp.float32),
        pltpu.VMEM((SPW, PACK * F), jnp.float32),
        pltpu.VMEM((BPW,), jnp.float32),
        pltpu.VMEM((BPW,), jnp.float32),
        pltpu.VMEM((L,), jnp.float32),
        pltpu.VMEM((BPW,), jnp.float32),
        pltpu.SemaphoreType.DMA,
    ],
)
def _mf_kernel(ulid_hbm, ilid_hbm, usub_hbm, isub_hbm, uid_hbm, iid_hbm,
               uf_hbm, if_hbm, ub_hbm, ib_hbm, gb_hbm,
               out_hbm,
               uid_v, iid_v, ulid_v, ilid_v, usub_v, isub_v, u_buf, i_buf,
               ub_v, ib_v, gb_v, out_v, sem):
    wid = lax.axis_index("s") * NC + lax.axis_index("c")
    base = wid * BPW

    pltpu.sync_copy(uid_hbm.at[pl.ds(base, BPW)], uid_v)
    pltpu.sync_copy(iid_hbm.at[pl.ds(base, BPW)], iid_v)
    pltpu.sync_copy(ulid_hbm.at[pl.ds(base, BPW)], ulid_v)
    pltpu.sync_copy(ilid_hbm.at[pl.ds(base, BPW)], ilid_v)
    pltpu.sync_copy(usub_hbm.at[pl.ds(base, BPW)], usub_v)
    pltpu.sync_copy(isub_hbm.at[pl.ds(base, BPW)], isub_v)

    cub = pltpu.async_copy(ub_hbm.at[uid_v], ub_v, sem)
    cib = pltpu.async_copy(ib_hbm.at[iid_v], ib_v, sem)
    pltpu.sync_copy(gb_hbm.at[...], gb_v)
    cub.wait()
    cib.wait()

    gb = gb_v[...]
    lane = lax.iota(jnp.int32, L)
    for s in range(STAGES):
        cu = pltpu.async_copy(
            uf_hbm.at[ulid_v.at[pl.ds(s * SPW, SPW)]], u_buf, sem)
        ci = pltpu.async_copy(
            if_hbm.at[ilid_v.at[pl.ds(s * SPW, SPW)]], i_buf, sem)
        cu.wait()
        ci.wait()
        for c in range(SCHUNKS):
            g = s * SPW + c * L
            lrows = jnp.full((L,), c * L, jnp.int32) + lane
            su = usub_v[pl.ds(g, L)]
            si = isub_v[pl.ds(g, L)]
            acc = ub_v[pl.ds(g, L)] + ib_v[pl.ds(g, L)] + gb
            for f in range(F):
                uv = plsc.load_gather(u_buf, [lrows, su + f])
                iv = plsc.load_gather(i_buf, [lrows, si + f])
                acc = acc + uv * iv
            out_v[pl.ds(g, L)] = acc

    pltpu.sync_copy(out_v, out_hbm.at[pl.ds(base, BPW)])


def kernel(user_ids, item_ids, user_factors, item_factors, user_bias,
           item_bias, global_bias):
    uid = user_ids.astype(jnp.int32)
    iid = item_ids.astype(jnp.int32)
    ulid = uid >> 2
    ilid = iid >> 2
    usub = (uid & 3) * F
    isub = (iid & 3) * F
    uf = user_factors.reshape(N_ROWS // PACK, F * PACK)
    itf = item_factors.reshape(N_ROWS // PACK, F * PACK)
    ub = user_bias.reshape(-1)
    ib = item_bias.reshape(-1)
    gb = jnp.broadcast_to(global_bias.astype(jnp.float32), (L,))
    return _mf_kernel(ulid, ilid, usub, isub, uid, iid, uf, itf, ub, ib, gb)


# speedup vs baseline: 1.0183x; 1.0183x over previous
import functools

import jax
import jax.numpy as jnp
from jax import lax
from jax.experimental import pallas as pl
from jax.experimental.pallas import tpu as pltpu
from jax.experimental.pallas import tpu_sc as plsc

B = 16384
F = 32
NC = 2
NS = 16
L = 16
NW = NC * NS
BPW = B // NW
CHUNKS = BPW // L

_mesh = plsc.VectorSubcoreMesh(core_axis_name="c", subcore_axis_name="s")


@functools.partial(
    pl.kernel,
    mesh=_mesh,
    out_type=jax.ShapeDtypeStruct((B,), jnp.float32),
    compiler_params=pltpu.CompilerParams(
        needs_layout_passes=False, use_tc_tiling_on_sc=False),
    scratch_types=[
        pltpu.VMEM((BPW,), jnp.int32),
        pltpu.VMEM((BPW,), jnp.int32),
        pltpu.VMEM((BPW, F), jnp.float32),
        pltpu.VMEM((BPW, F), jnp.float32),
        pltpu.VMEM((L * L,), jnp.float32),
        pltpu.VMEM((BPW,), jnp.float32),
        pltpu.SemaphoreType.DMA,
    ],
)
def _mf_kernel(uid_hbm, iid_hbm, uf_hbm, if_hbm,
               out_hbm,
               idx_u, idx_i, u_rows, i_rows, t_v, out_v, sem):
    wid = lax.axis_index("s") * NC + lax.axis_index("c")
    base = wid * BPW

    pltpu.sync_copy(uid_hbm.at[pl.ds(base, BPW)], idx_u)
    pltpu.sync_copy(iid_hbm.at[pl.ds(base, BPW)], idx_i)

    cu = pltpu.async_copy(uf_hbm.at[idx_u], u_rows, sem)
    ci = pltpu.async_copy(if_hbm.at[idx_i], i_rows, sem)
    cu.wait()
    ci.wait()

    lane = lax.iota(jnp.int32, L)
    col = lane * L
    for c in range(CHUNKS):
        for j in range(L):
            r = c * L + j
            p = (u_rows[r, 0:L] * i_rows[r, 0:L]
                 + u_rows[r, L:F] * i_rows[r, L:F])
            plsc.store_scatter(t_v, [col + j], p)
        acc = t_v[pl.ds(0, L)]
        for l in range(1, L):
            acc = acc + t_v[pl.ds(l * L, L)]
        out_v[pl.ds(c * L, L)] = acc

    pltpu.sync_copy(out_v, out_hbm.at[pl.ds(base, BPW)])


def kernel(user_ids, item_ids, user_factors, item_factors, user_bias,
           item_bias, global_bias):
    uid = user_ids.astype(jnp.int32)
    iid = item_ids.astype(jnp.int32)
    return _mf_kernel(uid, iid, user_factors, item_factors)
